# baseline (speedup 1.0000x reference)
import functools

import jax
import jax.numpy as jnp
from jax import lax
from jax.experimental import pallas as pl
from jax.experimental.pallas import tpu as pltpu
from jax.experimental.pallas import tpu_sc as plsc

N = 10000
E = 160000
D = 128
L = 4
NC = 10
NG = 1000
NUM_RUNS = 10
P = 2.0 / 11.0
NSEG = NUM_RUNS * N
ETOT = NUM_RUNS * E

NCORE = 2
NSUB = 16
RUNS_PC = NUM_RUNS // NCORE
SA = 624
G = 125
CPT = (E // NSUB) // G
IGRP = 8


def _stripe_copy(src_at, dst_at, s):
  @pl.when(s < NSUB - 1)
  def _():
    pltpu.sync_copy(src_at(s * SA, SA), dst_at(s * SA, SA))

  @pl.when(s == NSUB - 1)
  def _():
    pltpu.sync_copy(src_at((NSUB - 1) * SA, N - (NSUB - 1) * SA),
                    dst_at((NSUB - 1) * SA, N - (NSUB - 1) * SA))


def _sc_fast_body(xf_hbm, src2_hbm, dst2_hbm, out_hbm,
                  srcbuf, dstbuf, rba, rbb, acc, sema, semb, semsa, semsb):
  c = lax.axis_index("c")
  s = lax.axis_index("s")

  def gstart(k, buf, sem):
    pltpu.async_copy(xf_hbm.at[srcbuf.at[k]], buf, sem)

  def gwait(k, buf, sem):
    pltpu.make_async_copy(xf_hbm.at[srcbuf.at[k]], buf, sem).wait()

  def sstart(k, buf, sem):
    pltpu.async_copy(buf, acc.at[dstbuf.at[k]], sem, add=True)

  def swait(k, buf, sem):
    pltpu.make_async_copy(buf, acc.at[dstbuf.at[k]], sem).wait()

  def run_body(bi, _):
    r = bi * NCORE + c
    rbase = r * N

    _stripe_copy(lambda o, n: xf_hbm.at[pl.ds(rbase + o, n), :],
                 lambda o, n: acc.at[pl.ds(o, n), :], s)
    plsc.subcore_barrier()

    ib = r * (E // G) + s * CPT

    def igroup(gi, _):
      pltpu.sync_copy(src2_hbm.at[pl.ds(ib + gi * IGRP, IGRP), :], srcbuf)
      pltpu.sync_copy(dst2_hbm.at[pl.ds(ib + gi * IGRP, IGRP), :], dstbuf)

      gstart(0, rba, sema)
      gstart(1, rbb, semb)

      def chunk2(t, _):
        k = 2 * t
        gwait(k, rba, sema)
        sstart(k, rba, semsa)
        gwait(k + 1, rbb, semb)
        sstart(k + 1, rbb, semsb)

        @pl.when(k + 2 < IGRP)
        def _():
          swait(k, rba, semsa)
          gstart(k + 2, rba, sema)
          swait(k + 1, rbb, semsb)
          gstart(k + 3, rbb, semb)

        @pl.when(k + 2 >= IGRP)
        def _():
          swait(k, rba, semsa)
          swait(k + 1, rbb, semsb)
        return 0
      lax.fori_loop(0, IGRP // 2, chunk2, 0)
      return 0
    lax.fori_loop(0, CPT // IGRP, igroup, 0)

    plsc.subcore_barrier()
    _stripe_copy(lambda o, n: acc.at[pl.ds(o, n), :],
                 lambda o, n: out_hbm.at[pl.ds(rbase + o, n), :], s)
    plsc.subcore_barrier()
    return 0

  lax.fori_loop(0, RUNS_PC, run_body, 0)


_SC_MESH = plsc.VectorSubcoreMesh(core_axis_name="c", subcore_axis_name="s",
                                  num_cores=NCORE, num_subcores=NSUB)

_sc_fast = pl.kernel(
    _sc_fast_body,
    out_type=jax.ShapeDtypeStruct((NSEG, D), jnp.float32),
    mesh=_SC_MESH,
    scratch_types=[
        pltpu.VMEM((IGRP, G), jnp.int32),
        pltpu.VMEM((IGRP, G), jnp.int32),
        pltpu.VMEM((G, D), jnp.float32),
        pltpu.VMEM((G, D), jnp.float32),
        pltpu.VMEM_SHARED((N, D), jnp.float32),
        pltpu.SemaphoreType.DMA,
        pltpu.SemaphoreType.DMA,
        pltpu.SemaphoreType.DMA,
        pltpu.SemaphoreType.DMA,
    ],
)

RPB = 3200
NBPC = 16
FSTRIPE = RPB // NSUB
G2 = 128
FROWS = 12544
FRPT = FROWS // NSUB
FCB = 112
FCPT = FRPT // FCB


def _sc_gen_body(xf_hbm, src2_hbm, dst2_hbm, out_hbm,
                 srcbuf, dstbuf, rowbuf, acc, sem):
  c = lax.axis_index("c")
  s = lax.axis_index("s")

  def pass_body(bi, _):
    base = (bi * NCORE + c) * RPB
    slo = base + s * FSTRIPE

    @pl.when(slo + FSTRIPE <= NSEG)
    def _():
      pltpu.sync_copy(xf_hbm.at[pl.ds(slo, FSTRIPE), :],
                      acc.at[pl.ds(s * FSTRIPE, FSTRIPE), :])
    plsc.subcore_barrier()

    def block(t, _):
      ib = s * FRPT + t * FCB
      pltpu.sync_copy(src2_hbm.at[pl.ds(ib, FCB), :], srcbuf)
      pltpu.sync_copy(dst2_hbm.at[pl.ds(ib, FCB), :], dstbuf)

      def fix(k, _):
        def grp(j, _):
          sl = pl.ds(j * 16, 16)
          dv = dstbuf[k, sl]
          sv = srcbuf[k, sl]
          m = jnp.logical_and(dv >= base, dv < base + RPB)
          srcbuf[k, sl] = jnp.where(m, sv, jnp.int32(0))
          dstbuf[k, sl] = jnp.where(m, dv - base, jnp.int32(RPB))
          return 0
        lax.fori_loop(0, G2 // 16, grp, 0)
        return 0
      lax.fori_loop(0, FCB, fix, 0)

      def chunk(k, _):
        pltpu.async_copy(xf_hbm.at[srcbuf.at[k]], rowbuf, sem).wait()
        pltpu.sync_copy(rowbuf, acc.at[dstbuf.at[k]], add=True)
        return 0
      lax.fori_loop(0, FCB, chunk, 0)
      return 0
    lax.fori_loop(0, FCPT, block, 0)

    plsc.subcore_barrier()

    @pl.when(slo + FSTRIPE <= NSEG)
    def _():
      pltpu.sync_copy(acc.at[pl.ds(s * FSTRIPE, FSTRIPE), :],
                      out_hbm.at[pl.ds(slo, FSTRIPE), :])
    plsc.subcore_barrier()
    return 0

  lax.fori_loop(0, NBPC, pass_body, 0)


_sc_gen = pl.kernel(
    _sc_gen_body,
    out_type=jax.ShapeDtypeStruct((NSEG, D), jnp.float32),
    mesh=_SC_MESH,
    scratch_types=[
        pltpu.VMEM((FCB, G2), jnp.int32),
        pltpu.VMEM((FCB, G2), jnp.int32),
        pltpu.VMEM((G2, D), jnp.float32),
        pltpu.VMEM_SHARED((RPB + 8, D), jnp.float32),
        pltpu.SemaphoreType.DMA,
    ],
)


BR = 5000
NBLK = NSEG // BR
HP = jax.lax.Precision.HIGHEST


def _dot(a, b):
  return lax.dot_general(a, b, (((1,), (0,)), ((), ())),
                         precision=HP, preferred_element_type=jnp.float32)


def _k0_body(x_ref, keep_ref, fcw_ref, xr_ref, q_ref):
  pid = pl.program_id(0)
  xr = x_ref[...] * keep_ref[...]
  xr_ref[...] = xr
  contrib = _dot(xr, fcw_ref[...])
  nb = (pid % 2) * BR

  @pl.when(pid < 2)
  def _():
    q_ref[pl.ds(nb, BR), :] = contrib

  @pl.when(pid >= 2)
  def _():
    q_ref[pl.ds(nb, BR), :] = q_ref[pl.ds(nb, BR), :] + contrib


def _ka_body(h_ref, w_ref, b_ref, h1_ref, st_ref, acc):
  pid = pl.program_id(0)
  h1 = _dot(h_ref[...], w_ref[...]) + b_ref[...]
  h1_ref[...] = h1
  ssum = jnp.sum(h1, axis=0, keepdims=True)
  ssq = jnp.sum(h1 * h1, axis=0, keepdims=True)

  @pl.when(pid == 0)
  def _():
    acc[0:1, :] = ssum
    acc[1:2, :] = ssq

  @pl.when(pid > 0)
  def _():
    acc[0:1, :] = acc[0:1, :] + ssum
    acc[1:2, :] = acc[1:2, :] + ssq

  @pl.when(pid == NBLK - 1)
  def _():
    st_ref[...] = acc[...]


def _norm_relu(h, st, g, b):
  m = st[0:1, :] * (1.0 / NSEG)
  var = st[1:2, :] * (1.0 / NSEG) - m * m
  scale = lax.rsqrt(var + 1e-5) * g
  shift = b - m * scale
  return jnp.maximum(h * scale + shift, 0.0)


def _kb_body(h1_ref, st_ref, g_ref, bb_ref, w_ref, b2_ref, h2_ref, st2_ref,
             acc):
  pid = pl.program_id(0)
  hn = _norm_relu(h1_ref[...], st_ref[...], g_ref[...], bb_ref[...])
  h2 = _dot(hn, w_ref[...]) + b2_ref[...]
  h2_ref[...] = h2
  ssum = jnp.sum(h2, axis=0, keepdims=True)
  ssq = jnp.sum(h2 * h2, axis=0, keepdims=True)

  @pl.when(pid == 0)
  def _():
    acc[0:1, :] = ssum
    acc[1:2, :] = ssq

  @pl.when(pid > 0)
  def _():
    acc[0:1, :] = acc[0:1, :] + ssum
    acc[1:2, :] = acc[1:2, :] + ssq

  @pl.when(pid == NBLK - 1)
  def _():
    st2_ref[...] = acc[...]


def _kc_body(h2_ref, st_ref, g_ref, bb_ref, fcw_ref, xf_ref, q_ref):
  pid = pl.program_id(0)
  xf = _norm_relu(h2_ref[...], st_ref[...], g_ref[...], bb_ref[...])
  xf_ref[...] = xf
  contrib = _dot(xf, fcw_ref[...])
  nb = (pid % 2) * BR

  @pl.when(pid < 2)
  def _():
    q_ref[pl.ds(nb, BR), :] = contrib

  @pl.when(pid >= 2)
  def _():
    q_ref[pl.ds(nb, BR), :] = q_ref[pl.ds(nb, BR), :] + contrib


def _kf_body(q0, q1, q2, q3, q4, batch_ref, fcb_ref, out_ref, acc):
  pid = pl.program_id(0)
  qs = q0[...] + q1[...] + q2[...] + q3[...] + q4[...]
  bvals = batch_ref[0]
  giota = lax.broadcasted_iota(jnp.int32, (NG, BR), 0)
  oh = jnp.where(bvals == giota, 1.0, 0.0)
  part = lax.dot_general(oh, qs, (((1,), (0,)), ((), ())),
                         precision=HP, preferred_element_type=jnp.float32)

  @pl.when(pid == 0)
  def _():
    acc[...] = part

  @pl.when(pid == 1)
  def _():
    z = (acc[...] + part) * (1.0 / NUM_RUNS)
    z = z + jnp.sum(fcb_ref[...], axis=0, keepdims=True)
    mx = jnp.max(z, axis=1, keepdims=True)
    zz = z - mx
    out_ref[...] = zz - jnp.log(jnp.sum(jnp.exp(zz), axis=1, keepdims=True))


_f32 = jnp.float32


def _spec(bs, im):
  return pl.BlockSpec(bs, im)


_k0 = pl.pallas_call(
    _k0_body,
    grid=(NBLK,),
    in_specs=[
        _spec((BR, D), lambda i: (i % 2, 0)),
        _spec((BR, 1), lambda i: (i, 0)),
        _spec((D, NC), lambda i: (0, 0)),
    ],
    out_specs=[
        _spec((BR, D), lambda i: (i, 0)),
        _spec((N, NC), lambda i: (0, 0)),
    ],
    out_shape=[
        jax.ShapeDtypeStruct((NSEG, D), _f32),
        jax.ShapeDtypeStruct((N, NC), _f32),
    ],
)

_ka = pl.pallas_call(
    _ka_body,
    grid=(NBLK,),
    in_specs=[
        _spec((BR, D), lambda i: (i, 0)),
        _spec((D, D), lambda i: (0, 0)),
        _spec((1, D), lambda i: (0, 0)),
    ],
    out_specs=[
        _spec((BR, D), lambda i: (i, 0)),
        _spec((2, D), lambda i: (0, 0)),
    ],
    out_shape=[
        jax.ShapeDtypeStruct((NSEG, D), _f32),
        jax.ShapeDtypeStruct((2, D), _f32),
    ],
    scratch_shapes=[pltpu.VMEM((2, D), _f32)],
)

_kb = pl.pallas_call(
    _kb_body,
    grid=(NBLK,),
    in_specs=[
        _spec((BR, D), lambda i: (i, 0)),
        _spec((2, D), lambda i: (0, 0)),
        _spec((1, D), lambda i: (0, 0)),
        _spec((1, D), lambda i: (0, 0)),
        _spec((D, D), lambda i: (0, 0)),
        _spec((1, D), lambda i: (0, 0)),
    ],
    out_specs=[
        _spec((BR, D), lambda i: (i, 0)),
        _spec((2, D), lambda i: (0, 0)),
    ],
    out_shape=[
        jax.ShapeDtypeStruct((NSEG, D), _f32),
        jax.ShapeDtypeStruct((2, D), _f32),
    ],
    scratch_shapes=[pltpu.VMEM((2, D), _f32)],
)

_kc = pl.pallas_call(
    _kc_body,
    grid=(NBLK,),
    in_specs=[
        _spec((BR, D), lambda i: (i, 0)),
        _spec((2, D), lambda i: (0, 0)),
        _spec((1, D), lambda i: (0, 0)),
        _spec((1, D), lambda i: (0, 0)),
        _spec((D, NC), lambda i: (0, 0)),
    ],
    out_specs=[
        _spec((BR, D), lambda i: (i, 0)),
        _spec((N, NC), lambda i: (0, 0)),
    ],
    out_shape=[
        jax.ShapeDtypeStruct((NSEG, D), _f32),
        jax.ShapeDtypeStruct((N, NC), _f32),
    ],
)

_kf = pl.pallas_call(
    _kf_body,
    grid=(2,),
    in_specs=[
        _spec((BR, NC), lambda i: (i, 0)),
        _spec((BR, NC), lambda i: (i, 0)),
        _spec((BR, NC), lambda i: (i, 0)),
        _spec((BR, NC), lambda i: (i, 0)),
        _spec((BR, NC), lambda i: (i, 0)),
        _spec((1, 1, BR), lambda i: (i, 0, 0)),
        _spec((L + 1, NC), lambda i: (0, 0)),
    ],
    out_specs=_spec((NG, NC), lambda i: (0, 0)),
    out_shape=jax.ShapeDtypeStruct((NG, NC), _f32),
    scratch_shapes=[pltpu.VMEM((NG, NC), _f32)],
)


def kernel(x, edge_index, batch, convW1, convb1, conv_bn_g, conv_bn_b,
           convW2, convb2, bn_g, bn_b, fc_W, fc_b):
  drop = jax.random.bernoulli(jax.random.key(42), P, (NUM_RUNS, N))
  keep = jnp.where(drop, 0.0, 1.0).astype(jnp.float32).reshape(NSEG, 1)
  offset = edge_index.max() + 1
  run_off = jnp.arange(NUM_RUNS, dtype=edge_index.dtype) * offset
  srcf = (edge_index[0][None, :] + run_off[:, None]).reshape(-1)
  dstf = (edge_index[1][None, :] + run_off[:, None]).reshape(-1)
  src2 = srcf.reshape(ETOT // G, G)
  dstl2 = jnp.broadcast_to(edge_index[1][None, :],
                           (NUM_RUNS, E)).reshape(ETOT // G, G)
  npad = FROWS * G2 - ETOT
  src2g = jnp.concatenate(
      [srcf, jnp.zeros((npad,), jnp.int32)]).reshape(FROWS, G2)
  dst2g = jnp.concatenate(
      [dstf, jnp.full((npad,), -1, jnp.int32)]).reshape(FROWS, G2)

  def segsum(xf):
    return lax.cond(offset == N,
                    lambda a: _sc_fast(a, src2, dstl2),
                    lambda a: _sc_gen(a, src2g, dst2g),
                    xf)

  xr, q0 = _k0(x, keep, fc_W[0])
  qs = [q0]
  xf = xr
  for i in range(L):
    hsum = segsum(xf)
    h1, st1 = _ka(hsum, convW1[i], convb1[i][None, :])
    h2, st2 = _kb(h1, st1, conv_bn_g[i][None, :], conv_bn_b[i][None, :],
                  convW2[i], convb2[i][None, :])
    xf, q = _kc(h2, st2, bn_g[i][None, :], bn_b[i][None, :], fc_W[i + 1])
    qs.append(q)
  return _kf(*qs, batch.reshape(2, 1, BR), fc_b)


# speedup vs baseline: 1.3230x; 1.3230x over previous
import functools

import jax
import jax.numpy as jnp
from jax import lax
from jax.experimental import pallas as pl
from jax.experimental.pallas import tpu as pltpu
from jax.experimental.pallas import tpu_sc as plsc

N = 10000
E = 160000
D = 128
L = 4
NC = 10
NG = 1000
NUM_RUNS = 10
P = 2.0 / 11.0
NSEG = NUM_RUNS * N
ETOT = NUM_RUNS * E

NCORE = 2
NSUB = 16
RUNS_PC = NUM_RUNS // NCORE
SA = 624
G = 125
CPT = (E // NSUB) // G
IGRP = 8


def _stripe_copy(src_at, dst_at, s):
  @pl.when(s < NSUB - 1)
  def _():
    pltpu.sync_copy(src_at(s * SA, SA), dst_at(s * SA, SA))

  @pl.when(s == NSUB - 1)
  def _():
    pltpu.sync_copy(src_at((NSUB - 1) * SA, N - (NSUB - 1) * SA),
                    dst_at((NSUB - 1) * SA, N - (NSUB - 1) * SA))


def _sc_fast_body(xf_hbm, src2_hbm, dst2_hbm, out_hbm,
                  srcbuf, dstbuf, rba, rbb, acc, sema, semb):
  c = lax.axis_index("c")
  s = lax.axis_index("s")

  def gstart(k, buf, sem):
    pltpu.async_copy(xf_hbm.at[srcbuf.at[k]], buf, sem)

  def gwait(k, buf, sem):
    pltpu.make_async_copy(xf_hbm.at[srcbuf.at[k]], buf, sem).wait()

  def run_body(bi, _):
    r = bi * NCORE + c
    rbase = r * N

    _stripe_copy(lambda o, n: xf_hbm.at[pl.ds(rbase + o, n), :],
                 lambda o, n: acc.at[pl.ds(o, n), :], s)
    plsc.subcore_barrier()

    ib = r * (E // G) + s * CPT

    def igroup(gi, _):
      pltpu.sync_copy(src2_hbm.at[pl.ds(ib + gi * IGRP, IGRP), :], srcbuf)
      pltpu.sync_copy(dst2_hbm.at[pl.ds(ib + gi * IGRP, IGRP), :], dstbuf)

      gstart(0, rba, sema)

      def chunk2(t, _):
        k = 2 * t
        gstart(k + 1, rbb, semb)
        gwait(k, rba, sema)
        pltpu.sync_copy(rba, acc.at[dstbuf.at[k]], add=True)

        @pl.when(k + 2 < IGRP)
        def _():
          gstart(k + 2, rba, sema)
        gwait(k + 1, rbb, semb)
        pltpu.sync_copy(rbb, acc.at[dstbuf.at[k + 1]], add=True)
        return 0
      lax.fori_loop(0, IGRP // 2, chunk2, 0)
      return 0
    lax.fori_loop(0, CPT // IGRP, igroup, 0)

    plsc.subcore_barrier()
    _stripe_copy(lambda o, n: acc.at[pl.ds(o, n), :],
                 lambda o, n: out_hbm.at[pl.ds(rbase + o, n), :], s)
    plsc.subcore_barrier()
    return 0

  lax.fori_loop(0, RUNS_PC, run_body, 0)


_SC_MESH = plsc.VectorSubcoreMesh(core_axis_name="c", subcore_axis_name="s",
                                  num_cores=NCORE, num_subcores=NSUB)

_sc_fast = pl.kernel(
    _sc_fast_body,
    out_type=jax.ShapeDtypeStruct((NSEG, D), jnp.float32),
    mesh=_SC_MESH,
    scratch_types=[
        pltpu.VMEM((IGRP, G), jnp.int32),
        pltpu.VMEM((IGRP, G), jnp.int32),
        pltpu.VMEM((G, D), jnp.float32),
        pltpu.VMEM((G, D), jnp.float32),
        pltpu.VMEM_SHARED((N, D), jnp.float32),
        pltpu.SemaphoreType.DMA,
        pltpu.SemaphoreType.DMA,
    ],
)

RPB = 3200
NBPC = 16
FSTRIPE = RPB // NSUB
G2 = 128
FROWS = 12544
FRPT = FROWS // NSUB
FCB = 112
FCPT = FRPT // FCB


def _sc_gen_body(xf_hbm, src2_hbm, dst2_hbm, out_hbm,
                 srcbuf, dstbuf, rowbuf, acc, sem):
  c = lax.axis_index("c")
  s = lax.axis_index("s")

  def pass_body(bi, _):
    base = (bi * NCORE + c) * RPB
    slo = base + s * FSTRIPE

    @pl.when(slo + FSTRIPE <= NSEG)
    def _():
      pltpu.sync_copy(xf_hbm.at[pl.ds(slo, FSTRIPE), :],
                      acc.at[pl.ds(s * FSTRIPE, FSTRIPE), :])
    plsc.subcore_barrier()

    def block(t, _):
      ib = s * FRPT + t * FCB
      pltpu.sync_copy(src2_hbm.at[pl.ds(ib, FCB), :], srcbuf)
      pltpu.sync_copy(dst2_hbm.at[pl.ds(ib, FCB), :], dstbuf)

      def fix(k, _):
        def grp(j, _):
          sl = pl.ds(j * 16, 16)
          dv = dstbuf[k, sl]
          sv = srcbuf[k, sl]
          m = jnp.logical_and(dv >= base, dv < base + RPB)
          srcbuf[k, sl] = jnp.where(m, sv, jnp.int32(0))
          dstbuf[k, sl] = jnp.where(m, dv - base, jnp.int32(RPB))
          return 0
        lax.fori_loop(0, G2 // 16, grp, 0)
        return 0
      lax.fori_loop(0, FCB, fix, 0)

      def chunk(k, _):
        pltpu.async_copy(xf_hbm.at[srcbuf.at[k]], rowbuf, sem).wait()
        pltpu.sync_copy(rowbuf, acc.at[dstbuf.at[k]], add=True)
        return 0
      lax.fori_loop(0, FCB, chunk, 0)
      return 0
    lax.fori_loop(0, FCPT, block, 0)

    plsc.subcore_barrier()

    @pl.when(slo + FSTRIPE <= NSEG)
    def _():
      pltpu.sync_copy(acc.at[pl.ds(s * FSTRIPE, FSTRIPE), :],
                      out_hbm.at[pl.ds(slo, FSTRIPE), :])
    plsc.subcore_barrier()
    return 0

  lax.fori_loop(0, NBPC, pass_body, 0)


_sc_gen = pl.kernel(
    _sc_gen_body,
    out_type=jax.ShapeDtypeStruct((NSEG, D), jnp.float32),
    mesh=_SC_MESH,
    scratch_types=[
        pltpu.VMEM((FCB, G2), jnp.int32),
        pltpu.VMEM((FCB, G2), jnp.int32),
        pltpu.VMEM((G2, D), jnp.float32),
        pltpu.VMEM_SHARED((RPB + 8, D), jnp.float32),
        pltpu.SemaphoreType.DMA,
    ],
)


BR = 5000
NBLK = NSEG // BR
HP = jax.lax.Precision.DEFAULT


def _dot(a, b):
  return lax.dot_general(a, b, (((1,), (0,)), ((), ())),
                         precision=HP, preferred_element_type=jnp.float32)


def _k0_body(x_ref, keep_ref, fcw_ref, xr_ref, q_ref):
  pid = pl.program_id(0)
  xr = x_ref[...] * keep_ref[...]
  xr_ref[...] = xr
  contrib = _dot(xr, fcw_ref[...])
  nb = (pid % 2) * BR

  @pl.when(pid < 2)
  def _():
    q_ref[pl.ds(nb, BR), :] = contrib

  @pl.when(pid >= 2)
  def _():
    q_ref[pl.ds(nb, BR), :] = q_ref[pl.ds(nb, BR), :] + contrib


def _ka_body(h_ref, w_ref, b_ref, h1_ref, st_ref, acc):
  pid = pl.program_id(0)
  h1 = _dot(h_ref[...], w_ref[...]) + b_ref[...]
  h1_ref[...] = h1
  ssum = jnp.sum(h1, axis=0, keepdims=True)
  ssq = jnp.sum(h1 * h1, axis=0, keepdims=True)

  @pl.when(pid == 0)
  def _():
    acc[0:1, :] = ssum
    acc[1:2, :] = ssq

  @pl.when(pid > 0)
  def _():
    acc[0:1, :] = acc[0:1, :] + ssum
    acc[1:2, :] = acc[1:2, :] + ssq

  @pl.when(pid == NBLK - 1)
  def _():
    st_ref[...] = acc[...]


def _norm_relu(h, st, g, b):
  m = st[0:1, :] * (1.0 / NSEG)
  var = st[1:2, :] * (1.0 / NSEG) - m * m
  scale = lax.rsqrt(var + 1e-5) * g
  shift = b - m * scale
  return jnp.maximum(h * scale + shift, 0.0)


def _kb_body(h1_ref, st_ref, g_ref, bb_ref, w_ref, b2_ref, h2_ref, st2_ref,
             acc):
  pid = pl.program_id(0)
  hn = _norm_relu(h1_ref[...], st_ref[...], g_ref[...], bb_ref[...])
  h2 = _dot(hn, w_ref[...]) + b2_ref[...]
  h2_ref[...] = h2
  ssum = jnp.sum(h2, axis=0, keepdims=True)
  ssq = jnp.sum(h2 * h2, axis=0, keepdims=True)

  @pl.when(pid == 0)
  def _():
    acc[0:1, :] = ssum
    acc[1:2, :] = ssq

  @pl.when(pid > 0)
  def _():
    acc[0:1, :] = acc[0:1, :] + ssum
    acc[1:2, :] = acc[1:2, :] + ssq

  @pl.when(pid == NBLK - 1)
  def _():
    st2_ref[...] = acc[...]


def _kc_body(h2_ref, st_ref, g_ref, bb_ref, fcw_ref, xf_ref, q_ref):
  pid = pl.program_id(0)
  xf = _norm_relu(h2_ref[...], st_ref[...], g_ref[...], bb_ref[...])
  xf_ref[...] = xf
  contrib = _dot(xf, fcw_ref[...])
  nb = (pid % 2) * BR

  @pl.when(pid < 2)
  def _():
    q_ref[pl.ds(nb, BR), :] = contrib

  @pl.when(pid >= 2)
  def _():
    q_ref[pl.ds(nb, BR), :] = q_ref[pl.ds(nb, BR), :] + contrib


def _kf_body(q0, q1, q2, q3, q4, batch_ref, fcb_ref, out_ref, acc):
  pid = pl.program_id(0)
  qs = q0[...] + q1[...] + q2[...] + q3[...] + q4[...]
  bvals = batch_ref[0]
  giota = lax.broadcasted_iota(jnp.int32, (NG, BR), 0)
  oh = jnp.where(bvals == giota, 1.0, 0.0)
  part = lax.dot_general(oh, qs, (((1,), (0,)), ((), ())),
                         precision=HP, preferred_element_type=jnp.float32)

  @pl.when(pid == 0)
  def _():
    acc[...] = part

  @pl.when(pid == 1)
  def _():
    z = (acc[...] + part) * (1.0 / NUM_RUNS)
    z = z + jnp.sum(fcb_ref[...], axis=0, keepdims=True)
    mx = jnp.max(z, axis=1, keepdims=True)
    zz = z - mx
    out_ref[...] = zz - jnp.log(jnp.sum(jnp.exp(zz), axis=1, keepdims=True))


_f32 = jnp.float32


def _spec(bs, im):
  return pl.BlockSpec(bs, im)


_k0 = pl.pallas_call(
    _k0_body,
    grid=(NBLK,),
    in_specs=[
        _spec((BR, D), lambda i: (i % 2, 0)),
        _spec((BR, 1), lambda i: (i, 0)),
        _spec((D, NC), lambda i: (0, 0)),
    ],
    out_specs=[
        _spec((BR, D), lambda i: (i, 0)),
        _spec((N, NC), lambda i: (0, 0)),
    ],
    out_shape=[
        jax.ShapeDtypeStruct((NSEG, D), _f32),
        jax.ShapeDtypeStruct((N, NC), _f32),
    ],
)

_ka = pl.pallas_call(
    _ka_body,
    grid=(NBLK,),
    in_specs=[
        _spec((BR, D), lambda i: (i, 0)),
        _spec((D, D), lambda i: (0, 0)),
        _spec((1, D), lambda i: (0, 0)),
    ],
    out_specs=[
        _spec((BR, D), lambda i: (i, 0)),
        _spec((2, D), lambda i: (0, 0)),
    ],
    out_shape=[
        jax.ShapeDtypeStruct((NSEG, D), _f32),
        jax.ShapeDtypeStruct((2, D), _f32),
    ],
    scratch_shapes=[pltpu.VMEM((2, D), _f32)],
)

_kb = pl.pallas_call(
    _kb_body,
    grid=(NBLK,),
    in_specs=[
        _spec((BR, D), lambda i: (i, 0)),
        _spec((2, D), lambda i: (0, 0)),
        _spec((1, D), lambda i: (0, 0)),
        _spec((1, D), lambda i: (0, 0)),
        _spec((D, D), lambda i: (0, 0)),
        _spec((1, D), lambda i: (0, 0)),
    ],
    out_specs=[
        _spec((BR, D), lambda i: (i, 0)),
        _spec((2, D), lambda i: (0, 0)),
    ],
    out_shape=[
        jax.ShapeDtypeStruct((NSEG, D), _f32),
        jax.ShapeDtypeStruct((2, D), _f32),
    ],
    scratch_shapes=[pltpu.VMEM((2, D), _f32)],
)

_kc = pl.pallas_call(
    _kc_body,
    grid=(NBLK,),
    in_specs=[
        _spec((BR, D), lambda i: (i, 0)),
        _spec((2, D), lambda i: (0, 0)),
        _spec((1, D), lambda i: (0, 0)),
        _spec((1, D), lambda i: (0, 0)),
        _spec((D, NC), lambda i: (0, 0)),
    ],
    out_specs=[
        _spec((BR, D), lambda i: (i, 0)),
        _spec((N, NC), lambda i: (0, 0)),
    ],
    out_shape=[
        jax.ShapeDtypeStruct((NSEG, D), _f32),
        jax.ShapeDtypeStruct((N, NC), _f32),
    ],
)

_kf = pl.pallas_call(
    _kf_body,
    grid=(2,),
    in_specs=[
        _spec((BR, NC), lambda i: (i, 0)),
        _spec((BR, NC), lambda i: (i, 0)),
        _spec((BR, NC), lambda i: (i, 0)),
        _spec((BR, NC), lambda i: (i, 0)),
        _spec((BR, NC), lambda i: (i, 0)),
        _spec((1, 1, BR), lambda i: (i, 0, 0)),
        _spec((L + 1, NC), lambda i: (0, 0)),
    ],
    out_specs=_spec((NG, NC), lambda i: (0, 0)),
    out_shape=jax.ShapeDtypeStruct((NG, NC), _f32),
    scratch_shapes=[pltpu.VMEM((NG, NC), _f32)],
)


def kernel(x, edge_index, batch, convW1, convb1, conv_bn_g, conv_bn_b,
           convW2, convb2, bn_g, bn_b, fc_W, fc_b):
  drop = jax.random.bernoulli(jax.random.key(42), P, (NUM_RUNS, N))
  keep = jnp.where(drop, 0.0, 1.0).astype(jnp.float32).reshape(NSEG, 1)
  offset = edge_index.max() + 1
  run_off = jnp.arange(NUM_RUNS, dtype=edge_index.dtype) * offset
  srcf = (edge_index[0][None, :] + run_off[:, None]).reshape(-1)
  dstf = (edge_index[1][None, :] + run_off[:, None]).reshape(-1)
  src2 = srcf.reshape(ETOT // G, G)
  dstl2 = jnp.broadcast_to(edge_index[1][None, :],
                           (NUM_RUNS, E)).reshape(ETOT // G, G)
  npad = FROWS * G2 - ETOT
  src2g = jnp.concatenate(
      [srcf, jnp.zeros((npad,), jnp.int32)]).reshape(FROWS, G2)
  dst2g = jnp.concatenate(
      [dstf, jnp.full((npad,), -1, jnp.int32)]).reshape(FROWS, G2)

  def segsum(xf):
    return lax.cond(offset == N,
                    lambda a: _sc_fast(a, src2, dstl2),
                    lambda a: _sc_gen(a, src2g, dst2g),
                    xf)

  xr, q0 = _k0(x, keep, fc_W[0])
  qs = [q0]
  xf = xr
  for i in range(L):
    hsum = segsum(xf)
    h1, st1 = _ka(hsum, convW1[i], convb1[i][None, :])
    h2, st2 = _kb(h1, st1, conv_bn_g[i][None, :], conv_bn_b[i][None, :],
                  convW2[i], convb2[i][None, :])
    xf, q = _kc(h2, st2, bn_g[i][None, :], bn_b[i][None, :], fc_W[i + 1])
    qs.append(q)
  return _kf(*qs, batch.reshape(2, 1, BR), fc_b)


# speedup vs baseline: 1.4602x; 1.1037x over previous
import functools

import jax
import jax.numpy as jnp
from jax import lax
from jax.experimental import pallas as pl
from jax.experimental.pallas import tpu as pltpu
from jax.experimental.pallas import tpu_sc as plsc

N = 10000
E = 160000
D = 128
L = 4
NC = 10
NG = 1000
NUM_RUNS = 10
P = 2.0 / 11.0
NSEG = NUM_RUNS * N
ETOT = NUM_RUNS * E

NCORE = 2
NSUB = 16
RUNS_PC = NUM_RUNS // NCORE
SA = 624
G = 125
CPT = (E // NSUB) // G
IGRP = 16


def _stripe_copy(src_at, dst_at, s):
  @pl.when(s < NSUB - 1)
  def _():
    pltpu.sync_copy(src_at(s * SA, SA), dst_at(s * SA, SA))

  @pl.when(s == NSUB - 1)
  def _():
    pltpu.sync_copy(src_at((NSUB - 1) * SA, N - (NSUB - 1) * SA),
                    dst_at((NSUB - 1) * SA, N - (NSUB - 1) * SA))


def _sc_fast_body(xf_hbm, src2_hbm, dst2_hbm, out_hbm,
                  srcbuf, dstbuf, rba, rbb, acc, sema, semb):
  c = lax.axis_index("c")
  s = lax.axis_index("s")

  def gstart(k, buf, sem):
    pltpu.async_copy(xf_hbm.at[srcbuf.at[k]], buf, sem)

  def gwait(k, buf, sem):
    pltpu.make_async_copy(xf_hbm.at[srcbuf.at[k]], buf, sem).wait()

  def run_body(bi, _):
    r = bi * NCORE + c
    rbase = r * N

    _stripe_copy(lambda o, n: xf_hbm.at[pl.ds(rbase + o, n), :],
                 lambda o, n: acc.at[pl.ds(o, n), :], s)
    plsc.subcore_barrier()

    ib = r * (E // G) + s * CPT

    def igroup(gi, _):
      pltpu.sync_copy(src2_hbm.at[pl.ds(ib + gi * IGRP, IGRP), :], srcbuf)
      pltpu.sync_copy(dst2_hbm.at[pl.ds(ib + gi * IGRP, IGRP), :], dstbuf)

      gstart(0, rba, sema)

      def chunk2(t, _):
        k = 2 * t
        gstart(k + 1, rbb, semb)
        gwait(k, rba, sema)
        pltpu.sync_copy(rba, acc.at[dstbuf.at[k]], add=True)

        @pl.when(k + 2 < IGRP)
        def _():
          gstart(k + 2, rba, sema)
        gwait(k + 1, rbb, semb)
        pltpu.sync_copy(rbb, acc.at[dstbuf.at[k + 1]], add=True)
        return 0
      lax.fori_loop(0, IGRP // 2, chunk2, 0)
      return 0
    lax.fori_loop(0, CPT // IGRP, igroup, 0)

    plsc.subcore_barrier()
    _stripe_copy(lambda o, n: acc.at[pl.ds(o, n), :],
                 lambda o, n: out_hbm.at[pl.ds(rbase + o, n), :], s)
    plsc.subcore_barrier()
    return 0

  lax.fori_loop(0, RUNS_PC, run_body, 0)


_SC_MESH = plsc.VectorSubcoreMesh(core_axis_name="c", subcore_axis_name="s",
                                  num_cores=NCORE, num_subcores=NSUB)

_sc_fast = pl.kernel(
    _sc_fast_body,
    out_type=jax.ShapeDtypeStruct((NSEG, D), jnp.float32),
    mesh=_SC_MESH,
    scratch_types=[
        pltpu.VMEM((IGRP, G), jnp.int32),
        pltpu.VMEM((IGRP, G), jnp.int32),
        pltpu.VMEM((G, D), jnp.float32),
        pltpu.VMEM((G, D), jnp.float32),
        pltpu.VMEM_SHARED((N, D), jnp.float32),
        pltpu.SemaphoreType.DMA,
        pltpu.SemaphoreType.DMA,
    ],
)

RPB = 3200
NBPC = 16
FSTRIPE = RPB // NSUB
G2 = 128
FROWS = 12544
FRPT = FROWS // NSUB
FCB = 112
FCPT = FRPT // FCB


def _sc_gen_body(xf_hbm, src2_hbm, dst2_hbm, out_hbm,
                 srcbuf, dstbuf, rowbuf, acc, sem):
  c = lax.axis_index("c")
  s = lax.axis_index("s")

  def pass_body(bi, _):
    base = (bi * NCORE + c) * RPB
    slo = base + s * FSTRIPE

    @pl.when(slo + FSTRIPE <= NSEG)
    def _():
      pltpu.sync_copy(xf_hbm.at[pl.ds(slo, FSTRIPE), :],
                      acc.at[pl.ds(s * FSTRIPE, FSTRIPE), :])
    plsc.subcore_barrier()

    def block(t, _):
      ib = s * FRPT + t * FCB
      pltpu.sync_copy(src2_hbm.at[pl.ds(ib, FCB), :], srcbuf)
      pltpu.sync_copy(dst2_hbm.at[pl.ds(ib, FCB), :], dstbuf)

      def fix(k, _):
        def grp(j, _):
          sl = pl.ds(j * 16, 16)
          dv = dstbuf[k, sl]
          sv = srcbuf[k, sl]
          m = jnp.logical_and(dv >= base, dv < base + RPB)
          srcbuf[k, sl] = jnp.where(m, sv, jnp.int32(0))
          dstbuf[k, sl] = jnp.where(m, dv - base, jnp.int32(RPB))
          return 0
        lax.fori_loop(0, G2 // 16, grp, 0)
        return 0
      lax.fori_loop(0, FCB, fix, 0)

      def chunk(k, _):
        pltpu.async_copy(xf_hbm.at[srcbuf.at[k]], rowbuf, sem).wait()
        pltpu.sync_copy(rowbuf, acc.at[dstbuf.at[k]], add=True)
        return 0
      lax.fori_loop(0, FCB, chunk, 0)
      return 0
    lax.fori_loop(0, FCPT, block, 0)

    plsc.subcore_barrier()

    @pl.when(slo + FSTRIPE <= NSEG)
    def _():
      pltpu.sync_copy(acc.at[pl.ds(s * FSTRIPE, FSTRIPE), :],
                      out_hbm.at[pl.ds(slo, FSTRIPE), :])
    plsc.subcore_barrier()
    return 0

  lax.fori_loop(0, NBPC, pass_body, 0)


_sc_gen = pl.kernel(
    _sc_gen_body,
    out_type=jax.ShapeDtypeStruct((NSEG, D), jnp.float32),
    mesh=_SC_MESH,
    scratch_types=[
        pltpu.VMEM((FCB, G2), jnp.int32),
        pltpu.VMEM((FCB, G2), jnp.int32),
        pltpu.VMEM((G2, D), jnp.float32),
        pltpu.VMEM_SHARED((RPB + 8, D), jnp.float32),
        pltpu.SemaphoreType.DMA,
    ],
)


BR = 10000
NBLK = NSEG // BR
PBR = 5000
HP = jax.lax.Precision.DEFAULT


def _dot(a, b):
  return lax.dot_general(a, b, (((1,), (0,)), ((), ())),
                         precision=HP, preferred_element_type=jnp.float32)


def _k0_body(x_ref, keep_ref, fcw_ref, xr_ref, q_ref):
  pid = pl.program_id(0)
  xr = x_ref[...] * keep_ref[...]
  xr_ref[...] = xr
  contrib = _dot(xr, fcw_ref[...])

  @pl.when(pid == 0)
  def _():
    q_ref[...] = contrib

  @pl.when(pid > 0)
  def _():
    q_ref[...] = q_ref[...] + contrib


def _ka_body(h_ref, w_ref, b_ref, h1_ref, st_ref, acc):
  pid = pl.program_id(0)
  h1 = _dot(h_ref[...], w_ref[...]) + b_ref[...]
  h1_ref[...] = h1
  ssum = jnp.sum(h1, axis=0, keepdims=True)
  ssq = jnp.sum(h1 * h1, axis=0, keepdims=True)

  @pl.when(pid == 0)
  def _():
    acc[0:1, :] = ssum
    acc[1:2, :] = ssq

  @pl.when(pid > 0)
  def _():
    acc[0:1, :] = acc[0:1, :] + ssum
    acc[1:2, :] = acc[1:2, :] + ssq

  @pl.when(pid == NBLK - 1)
  def _():
    st_ref[...] = acc[...]


def _norm_relu(h, st, g, b):
  m = st[0:1, :] * (1.0 / NSEG)
  var = st[1:2, :] * (1.0 / NSEG) - m * m
  scale = lax.rsqrt(var + 1e-5) * g
  shift = b - m * scale
  return jnp.maximum(h * scale + shift, 0.0)


def _kb_body(h1_ref, st_ref, g_ref, bb_ref, w_ref, b2_ref, h2_ref, st2_ref,
             acc):
  pid = pl.program_id(0)
  hn = _norm_relu(h1_ref[...], st_ref[...], g_ref[...], bb_ref[...])
  h2 = _dot(hn, w_ref[...]) + b2_ref[...]
  h2_ref[...] = h2
  ssum = jnp.sum(h2, axis=0, keepdims=True)
  ssq = jnp.sum(h2 * h2, axis=0, keepdims=True)

  @pl.when(pid == 0)
  def _():
    acc[0:1, :] = ssum
    acc[1:2, :] = ssq

  @pl.when(pid > 0)
  def _():
    acc[0:1, :] = acc[0:1, :] + ssum
    acc[1:2, :] = acc[1:2, :] + ssq

  @pl.when(pid == NBLK - 1)
  def _():
    st2_ref[...] = acc[...]


def _kc_body(h2_ref, st_ref, g_ref, bb_ref, fcw_ref, xf_ref, q_ref):
  pid = pl.program_id(0)
  xf = _norm_relu(h2_ref[...], st_ref[...], g_ref[...], bb_ref[...])
  xf_ref[...] = xf
  contrib = _dot(xf, fcw_ref[...])

  @pl.when(pid == 0)
  def _():
    q_ref[...] = contrib

  @pl.when(pid > 0)
  def _():
    q_ref[...] = q_ref[...] + contrib


def _kf_body(q0, q1, q2, q3, q4, batch_ref, fcb_ref, out_ref, acc):
  pid = pl.program_id(0)
  qs = q0[...] + q1[...] + q2[...] + q3[...] + q4[...]
  bvals = batch_ref[0]
  giota = lax.broadcasted_iota(jnp.int32, (NG, PBR), 0)
  oh = jnp.where(bvals == giota, 1.0, 0.0)
  part = lax.dot_general(oh, qs, (((1,), (0,)), ((), ())),
                         precision=HP, preferred_element_type=jnp.float32)

  @pl.when(pid == 0)
  def _():
    acc[...] = part

  @pl.when(pid == 1)
  def _():
    z = (acc[...] + part) * (1.0 / NUM_RUNS)
    z = z + jnp.sum(fcb_ref[...], axis=0, keepdims=True)
    mx = jnp.max(z, axis=1, keepdims=True)
    zz = z - mx
    out_ref[...] = zz - jnp.log(jnp.sum(jnp.exp(zz), axis=1, keepdims=True))


_f32 = jnp.float32


def _spec(bs, im):
  return pl.BlockSpec(bs, im)


_k0 = pl.pallas_call(
    _k0_body,
    grid=(NBLK,),
    in_specs=[
        _spec((N, D), lambda i: (0, 0)),
        _spec((BR, 1), lambda i: (i, 0)),
        _spec((D, NC), lambda i: (0, 0)),
    ],
    out_specs=[
        _spec((BR, D), lambda i: (i, 0)),
        _spec((N, NC), lambda i: (0, 0)),
    ],
    out_shape=[
        jax.ShapeDtypeStruct((NSEG, D), _f32),
        jax.ShapeDtypeStruct((N, NC), _f32),
    ],
)

_ka = pl.pallas_call(
    _ka_body,
    grid=(NBLK,),
    in_specs=[
        _spec((BR, D), lambda i: (i, 0)),
        _spec((D, D), lambda i: (0, 0)),
        _spec((1, D), lambda i: (0, 0)),
    ],
    out_specs=[
        _spec((BR, D), lambda i: (i, 0)),
        _spec((2, D), lambda i: (0, 0)),
    ],
    out_shape=[
        jax.ShapeDtypeStruct((NSEG, D), _f32),
        jax.ShapeDtypeStruct((2, D), _f32),
    ],
    scratch_shapes=[pltpu.VMEM((2, D), _f32)],
)

_kb = pl.pallas_call(
    _kb_body,
    grid=(NBLK,),
    in_specs=[
        _spec((BR, D), lambda i: (i, 0)),
        _spec((2, D), lambda i: (0, 0)),
        _spec((1, D), lambda i: (0, 0)),
        _spec((1, D), lambda i: (0, 0)),
        _spec((D, D), lambda i: (0, 0)),
        _spec((1, D), lambda i: (0, 0)),
    ],
    out_specs=[
        _spec((BR, D), lambda i: (i, 0)),
        _spec((2, D), lambda i: (0, 0)),
    ],
    out_shape=[
        jax.ShapeDtypeStruct((NSEG, D), _f32),
        jax.ShapeDtypeStruct((2, D), _f32),
    ],
    scratch_shapes=[pltpu.VMEM((2, D), _f32)],
)

_kc = pl.pallas_call(
    _kc_body,
    grid=(NBLK,),
    in_specs=[
        _spec((BR, D), lambda i: (i, 0)),
        _spec((2, D), lambda i: (0, 0)),
        _spec((1, D), lambda i: (0, 0)),
        _spec((1, D), lambda i: (0, 0)),
        _spec((D, NC), lambda i: (0, 0)),
    ],
    out_specs=[
        _spec((BR, D), lambda i: (i, 0)),
        _spec((N, NC), lambda i: (0, 0)),
    ],
    out_shape=[
        jax.ShapeDtypeStruct((NSEG, D), _f32),
        jax.ShapeDtypeStruct((N, NC), _f32),
    ],
)

_kf = pl.pallas_call(
    _kf_body,
    grid=(2,),
    in_specs=[
        _spec((PBR, NC), lambda i: (i, 0)),
        _spec((PBR, NC), lambda i: (i, 0)),
        _spec((PBR, NC), lambda i: (i, 0)),
        _spec((PBR, NC), lambda i: (i, 0)),
        _spec((PBR, NC), lambda i: (i, 0)),
        _spec((1, 1, PBR), lambda i: (i, 0, 0)),
        _spec((L + 1, NC), lambda i: (0, 0)),
    ],
    out_specs=_spec((NG, NC), lambda i: (0, 0)),
    out_shape=jax.ShapeDtypeStruct((NG, NC), _f32),
    scratch_shapes=[pltpu.VMEM((NG, NC), _f32)],
)


def kernel(x, edge_index, batch, convW1, convb1, conv_bn_g, conv_bn_b,
           convW2, convb2, bn_g, bn_b, fc_W, fc_b):
  drop = jax.random.bernoulli(jax.random.key(42), P, (NUM_RUNS, N))
  keep = jnp.where(drop, 0.0, 1.0).astype(jnp.float32).reshape(NSEG, 1)
  offset = edge_index.max() + 1
  run_off = jnp.arange(NUM_RUNS, dtype=edge_index.dtype) * offset
  srcf = (edge_index[0][None, :] + run_off[:, None]).reshape(-1)
  dstf = (edge_index[1][None, :] + run_off[:, None]).reshape(-1)
  src2 = srcf.reshape(ETOT // G, G)
  dstl2 = jnp.broadcast_to(edge_index[1][None, :],
                           (NUM_RUNS, E)).reshape(ETOT // G, G)
  npad = FROWS * G2 - ETOT
  src2g = jnp.concatenate(
      [srcf, jnp.zeros((npad,), jnp.int32)]).reshape(FROWS, G2)
  dst2g = jnp.concatenate(
      [dstf, jnp.full((npad,), -1, jnp.int32)]).reshape(FROWS, G2)

  def segsum(xf):
    return lax.cond(offset == N,
                    lambda a: _sc_fast(a, src2, dstl2),
                    lambda a: _sc_gen(a, src2g, dst2g),
                    xf)

  xr, q0 = _k0(x, keep, fc_W[0])
  qs = [q0]
  xf = xr
  for i in range(L):
    hsum = segsum(xf)
    h1, st1 = _ka(hsum, convW1[i], convb1[i][None, :])
    h2, st2 = _kb(h1, st1, conv_bn_g[i][None, :], conv_bn_b[i][None, :],
                  convW2[i], convb2[i][None, :])
    xf, q = _kc(h2, st2, bn_g[i][None, :], bn_b[i][None, :], fc_W[i + 1])
    qs.append(q)
  return _kf(*qs, batch.reshape(2, 1, PBR), fc_b)


# speedup vs baseline: 1.5699x; 1.0752x over previous
import functools

import jax
import jax.numpy as jnp
from jax import lax
from jax.experimental import pallas as pl
from jax.experimental.pallas import tpu as pltpu
from jax.experimental.pallas import tpu_sc as plsc

N = 10000
E = 160000
D = 128
L = 4
NC = 10
NG = 1000
NUM_RUNS = 10
P = 2.0 / 11.0
NSEG = NUM_RUNS * N
ETOT = NUM_RUNS * E

NCORE = 2
NSUB = 16
RUNS_PC = NUM_RUNS // NCORE
SA = 624
G = 125
CPT = (E // NSUB) // G
IGRP = 40


def _stripe_copy(src_at, dst_at, s):
  @pl.when(s < NSUB - 1)
  def _():
    pltpu.sync_copy(src_at(s * SA, SA), dst_at(s * SA, SA))

  @pl.when(s == NSUB - 1)
  def _():
    pltpu.sync_copy(src_at((NSUB - 1) * SA, N - (NSUB - 1) * SA),
                    dst_at((NSUB - 1) * SA, N - (NSUB - 1) * SA))


def _sc_fast_body(xf_hbm, src2_hbm, dst2_hbm, out_hbm,
                  srcbuf, dstbuf, rba, rbb, acc, sema, semb):
  c = lax.axis_index("c")
  s = lax.axis_index("s")

  def gstart(k, buf, sem):
    pltpu.async_copy(xf_hbm.at[srcbuf.at[k]], buf, sem)

  def gwait(k, buf, sem):
    pltpu.make_async_copy(xf_hbm.at[srcbuf.at[k]], buf, sem).wait()

  def run_body(bi, _):
    r = bi * NCORE + c
    rbase = r * N

    _stripe_copy(lambda o, n: xf_hbm.at[pl.ds(rbase + o, n), :],
                 lambda o, n: acc.at[pl.ds(o, n), :], s)
    plsc.subcore_barrier()

    ib = r * (E // G) + s * CPT

    def igroup(gi, _):
      pltpu.sync_copy(src2_hbm.at[pl.ds(ib + gi * IGRP, IGRP), :], srcbuf)
      pltpu.sync_copy(dst2_hbm.at[pl.ds(ib + gi * IGRP, IGRP), :], dstbuf)

      gstart(0, rba, sema)

      def chunk2(t, _):
        k = 2 * t
        gstart(k + 1, rbb, semb)
        gwait(k, rba, sema)
        pltpu.sync_copy(rba, acc.at[dstbuf.at[k]], add=True)

        @pl.when(k + 2 < IGRP)
        def _():
          gstart(k + 2, rba, sema)
        gwait(k + 1, rbb, semb)
        pltpu.sync_copy(rbb, acc.at[dstbuf.at[k + 1]], add=True)
        return 0
      lax.fori_loop(0, IGRP // 2, chunk2, 0)
      return 0
    lax.fori_loop(0, CPT // IGRP, igroup, 0)

    plsc.subcore_barrier()
    _stripe_copy(lambda o, n: acc.at[pl.ds(o, n), :],
                 lambda o, n: out_hbm.at[pl.ds(rbase + o, n), :], s)
    return 0

  lax.fori_loop(0, RUNS_PC, run_body, 0)


_SC_MESH = plsc.VectorSubcoreMesh(core_axis_name="c", subcore_axis_name="s",
                                  num_cores=NCORE, num_subcores=NSUB)

_sc_fast = pl.kernel(
    _sc_fast_body,
    out_type=jax.ShapeDtypeStruct((NSEG, D), jnp.float32),
    mesh=_SC_MESH,
    scratch_types=[
        pltpu.VMEM((IGRP, G), jnp.int32),
        pltpu.VMEM((IGRP, G), jnp.int32),
        pltpu.VMEM((G, D), jnp.float32),
        pltpu.VMEM((G, D), jnp.float32),
        pltpu.VMEM_SHARED((N, D), jnp.float32),
        pltpu.SemaphoreType.DMA,
        pltpu.SemaphoreType.DMA,
    ],
)

RPB = 3200
NBPC = 16
FSTRIPE = RPB // NSUB
G2 = 128
FROWS = 12544
FRPT = FROWS // NSUB
FCB = 112
FCPT = FRPT // FCB


def _sc_gen_body(xf_hbm, src2_hbm, dst2_hbm, out_hbm,
                 srcbuf, dstbuf, rowbuf, acc, sem):
  c = lax.axis_index("c")
  s = lax.axis_index("s")

  def pass_body(bi, _):
    base = (bi * NCORE + c) * RPB
    slo = base + s * FSTRIPE

    @pl.when(slo + FSTRIPE <= NSEG)
    def _():
      pltpu.sync_copy(xf_hbm.at[pl.ds(slo, FSTRIPE), :],
                      acc.at[pl.ds(s * FSTRIPE, FSTRIPE), :])
    plsc.subcore_barrier()

    def block(t, _):
      ib = s * FRPT + t * FCB
      pltpu.sync_copy(src2_hbm.at[pl.ds(ib, FCB), :], srcbuf)
      pltpu.sync_copy(dst2_hbm.at[pl.ds(ib, FCB), :], dstbuf)

      def fix(k, _):
        def grp(j, _):
          sl = pl.ds(j * 16, 16)
          dv = dstbuf[k, sl]
          sv = srcbuf[k, sl]
          m = jnp.logical_and(dv >= base, dv < base + RPB)
          srcbuf[k, sl] = jnp.where(m, sv, jnp.int32(0))
          dstbuf[k, sl] = jnp.where(m, dv - base, jnp.int32(RPB))
          return 0
        lax.fori_loop(0, G2 // 16, grp, 0)
        return 0
      lax.fori_loop(0, FCB, fix, 0)

      def chunk(k, _):
        pltpu.async_copy(xf_hbm.at[srcbuf.at[k]], rowbuf, sem).wait()
        pltpu.sync_copy(rowbuf, acc.at[dstbuf.at[k]], add=True)
        return 0
      lax.fori_loop(0, FCB, chunk, 0)
      return 0
    lax.fori_loop(0, FCPT, block, 0)

    plsc.subcore_barrier()

    @pl.when(slo + FSTRIPE <= NSEG)
    def _():
      pltpu.sync_copy(acc.at[pl.ds(s * FSTRIPE, FSTRIPE), :],
                      out_hbm.at[pl.ds(slo, FSTRIPE), :])
    plsc.subcore_barrier()
    return 0

  lax.fori_loop(0, NBPC, pass_body, 0)


_sc_gen = pl.kernel(
    _sc_gen_body,
    out_type=jax.ShapeDtypeStruct((NSEG, D), jnp.float32),
    mesh=_SC_MESH,
    scratch_types=[
        pltpu.VMEM((FCB, G2), jnp.int32),
        pltpu.VMEM((FCB, G2), jnp.int32),
        pltpu.VMEM((G2, D), jnp.float32),
        pltpu.VMEM_SHARED((RPB + 8, D), jnp.float32),
        pltpu.SemaphoreType.DMA,
    ],
)


BR = 10000
NBLK = NSEG // BR
PBR = 5000
HP = jax.lax.Precision.DEFAULT


def _dot(a, b):
  return lax.dot_general(a, b, (((1,), (0,)), ((), ())),
                         precision=HP, preferred_element_type=jnp.float32)


def _k0_body(x_ref, keep_ref, fcw_ref, xr_ref, q_ref):
  pid = pl.program_id(0)
  xr = x_ref[...] * keep_ref[...]
  xr_ref[...] = xr
  contrib = _dot(xr, fcw_ref[...])

  @pl.when(pid == 0)
  def _():
    q_ref[...] = contrib

  @pl.when(pid > 0)
  def _():
    q_ref[...] = q_ref[...] + contrib


def _ka_body(h_ref, w_ref, b_ref, h1_ref, st_ref, acc):
  pid = pl.program_id(0)
  h1 = _dot(h_ref[...], w_ref[...]) + b_ref[...]
  h1_ref[...] = h1
  ssum = jnp.sum(h1, axis=0, keepdims=True)
  ssq = jnp.sum(h1 * h1, axis=0, keepdims=True)

  @pl.when(pid == 0)
  def _():
    acc[0:1, :] = ssum
    acc[1:2, :] = ssq

  @pl.when(pid > 0)
  def _():
    acc[0:1, :] = acc[0:1, :] + ssum
    acc[1:2, :] = acc[1:2, :] + ssq

  @pl.when(pid == NBLK - 1)
  def _():
    st_ref[...] = acc[...]


def _norm_relu(h, st, g, b):
  m = st[0:1, :] * (1.0 / NSEG)
  var = st[1:2, :] * (1.0 / NSEG) - m * m
  scale = lax.rsqrt(var + 1e-5) * g
  shift = b - m * scale
  return jnp.maximum(h * scale + shift, 0.0)


def _kb_body(h1_ref, st_ref, g_ref, bb_ref, w_ref, b2_ref, h2_ref, st2_ref,
             acc):
  pid = pl.program_id(0)
  hn = _norm_relu(h1_ref[...], st_ref[...], g_ref[...], bb_ref[...])
  h2 = _dot(hn, w_ref[...]) + b2_ref[...]
  h2_ref[...] = h2
  ssum = jnp.sum(h2, axis=0, keepdims=True)
  ssq = jnp.sum(h2 * h2, axis=0, keepdims=True)

  @pl.when(pid == 0)
  def _():
    acc[0:1, :] = ssum
    acc[1:2, :] = ssq

  @pl.when(pid > 0)
  def _():
    acc[0:1, :] = acc[0:1, :] + ssum
    acc[1:2, :] = acc[1:2, :] + ssq

  @pl.when(pid == NBLK - 1)
  def _():
    st2_ref[...] = acc[...]


def _kc_body(h2_ref, st_ref, g_ref, bb_ref, fcw_ref, xf_ref, q_ref):
  pid = pl.program_id(0)
  xf = _norm_relu(h2_ref[...], st_ref[...], g_ref[...], bb_ref[...])
  xf_ref[...] = xf
  contrib = _dot(xf, fcw_ref[...])

  @pl.when(pid == 0)
  def _():
    q_ref[...] = contrib

  @pl.when(pid > 0)
  def _():
    q_ref[...] = q_ref[...] + contrib


def _kf_body(q0, q1, q2, q3, q4, batch_ref, fcb_ref, out_ref, acc):
  pid = pl.program_id(0)
  qs = q0[...] + q1[...] + q2[...] + q3[...] + q4[...]
  bvals = batch_ref[0]
  giota = lax.broadcasted_iota(jnp.int32, (NG, PBR), 0)
  oh = jnp.where(bvals == giota, 1.0, 0.0)
  part = lax.dot_general(oh, qs, (((1,), (0,)), ((), ())),
                         precision=HP, preferred_element_type=jnp.float32)

  @pl.when(pid == 0)
  def _():
    acc[...] = part

  @pl.when(pid == 1)
  def _():
    z = (acc[...] + part) * (1.0 / NUM_RUNS)
    z = z + jnp.sum(fcb_ref[...], axis=0, keepdims=True)
    mx = jnp.max(z, axis=1, keepdims=True)
    zz = z - mx
    out_ref[...] = zz - jnp.log(jnp.sum(jnp.exp(zz), axis=1, keepdims=True))


_f32 = jnp.float32


def _spec(bs, im):
  return pl.BlockSpec(bs, im)


_k0 = pl.pallas_call(
    _k0_body,
    grid=(NBLK,),
    in_specs=[
        _spec((N, D), lambda i: (0, 0)),
        _spec((BR, 1), lambda i: (i, 0)),
        _spec((D, NC), lambda i: (0, 0)),
    ],
    out_specs=[
        _spec((BR, D), lambda i: (i, 0)),
        _spec((N, NC), lambda i: (0, 0)),
    ],
    out_shape=[
        jax.ShapeDtypeStruct((NSEG, D), _f32),
        jax.ShapeDtypeStruct((N, NC), _f32),
    ],
)

_ka = pl.pallas_call(
    _ka_body,
    grid=(NBLK,),
    in_specs=[
        _spec((BR, D), lambda i: (i, 0)),
        _spec((D, D), lambda i: (0, 0)),
        _spec((1, D), lambda i: (0, 0)),
    ],
    out_specs=[
        _spec((BR, D), lambda i: (i, 0)),
        _spec((2, D), lambda i: (0, 0)),
    ],
    out_shape=[
        jax.ShapeDtypeStruct((NSEG, D), _f32),
        jax.ShapeDtypeStruct((2, D), _f32),
    ],
    scratch_shapes=[pltpu.VMEM((2, D), _f32)],
)

_kb = pl.pallas_call(
    _kb_body,
    grid=(NBLK,),
    in_specs=[
        _spec((BR, D), lambda i: (i, 0)),
        _spec((2, D), lambda i: (0, 0)),
        _spec((1, D), lambda i: (0, 0)),
        _spec((1, D), lambda i: (0, 0)),
        _spec((D, D), lambda i: (0, 0)),
        _spec((1, D), lambda i: (0, 0)),
    ],
    out_specs=[
        _spec((BR, D), lambda i: (i, 0)),
        _spec((2, D), lambda i: (0, 0)),
    ],
    out_shape=[
        jax.ShapeDtypeStruct((NSEG, D), _f32),
        jax.ShapeDtypeStruct((2, D), _f32),
    ],
    scratch_shapes=[pltpu.VMEM((2, D), _f32)],
)

_kc = pl.pallas_call(
    _kc_body,
    grid=(NBLK,),
    in_specs=[
        _spec((BR, D), lambda i: (i, 0)),
        _spec((2, D), lambda i: (0, 0)),
        _spec((1, D), lambda i: (0, 0)),
        _spec((1, D), lambda i: (0, 0)),
        _spec((D, NC), lambda i: (0, 0)),
    ],
    out_specs=[
        _spec((BR, D), lambda i: (i, 0)),
        _spec((N, NC), lambda i: (0, 0)),
    ],
    out_shape=[
        jax.ShapeDtypeStruct((NSEG, D), _f32),
        jax.ShapeDtypeStruct((N, NC), _f32),
    ],
)

_kf = pl.pallas_call(
    _kf_body,
    grid=(2,),
    in_specs=[
        _spec((PBR, NC), lambda i: (i, 0)),
        _spec((PBR, NC), lambda i: (i, 0)),
        _spec((PBR, NC), lambda i: (i, 0)),
        _spec((PBR, NC), lambda i: (i, 0)),
        _spec((PBR, NC), lambda i: (i, 0)),
        _spec((1, 1, PBR), lambda i: (i, 0, 0)),
        _spec((L + 1, NC), lambda i: (0, 0)),
    ],
    out_specs=_spec((NG, NC), lambda i: (0, 0)),
    out_shape=jax.ShapeDtypeStruct((NG, NC), _f32),
    scratch_shapes=[pltpu.VMEM((NG, NC), _f32)],
)


def kernel(x, edge_index, batch, convW1, convb1, conv_bn_g, conv_bn_b,
           convW2, convb2, bn_g, bn_b, fc_W, fc_b):
  drop = jax.random.bernoulli(jax.random.key(42), P, (NUM_RUNS, N))
  keep = jnp.where(drop, 0.0, 1.0).astype(jnp.float32).reshape(NSEG, 1)
  offset = edge_index.max() + 1
  run_off = jnp.arange(NUM_RUNS, dtype=edge_index.dtype) * offset
  srcf = (edge_index[0][None, :] + run_off[:, None]).reshape(-1)
  dstf = (edge_index[1][None, :] + run_off[:, None]).reshape(-1)
  src2 = srcf.reshape(ETOT // G, G)
  dstl2 = jnp.broadcast_to(edge_index[1][None, :],
                           (NUM_RUNS, E)).reshape(ETOT // G, G)
  npad = FROWS * G2 - ETOT
  src2g = jnp.concatenate(
      [srcf, jnp.zeros((npad,), jnp.int32)]).reshape(FROWS, G2)
  dst2g = jnp.concatenate(
      [dstf, jnp.full((npad,), -1, jnp.int32)]).reshape(FROWS, G2)

  def segsum(xf):
    return lax.cond(offset == N,
                    lambda a: _sc_fast(a, src2, dstl2),
                    lambda a: _sc_gen(a, src2g, dst2g),
                    xf)

  xr, q0 = _k0(x, keep, fc_W[0])
  qs = [q0]
  xf = xr
  for i in range(L):
    hsum = segsum(xf)
    h1, st1 = _ka(hsum, convW1[i], convb1[i][None, :])
    h2, st2 = _kb(h1, st1, conv_bn_g[i][None, :], conv_bn_b[i][None, :],
                  convW2[i], convb2[i][None, :])
    xf, q = _kc(h2, st2, bn_g[i][None, :], bn_b[i][None, :], fc_W[i + 1])
    qs.append(q)
  return _kf(*qs, batch.reshape(2, 1, PBR), fc_b)


# speedup vs baseline: 1.6253x; 1.0353x over previous
import functools

import jax
import jax.numpy as jnp
from jax import lax
from jax.experimental import pallas as pl
from jax.experimental.pallas import tpu as pltpu
from jax.experimental.pallas import tpu_sc as plsc

N = 10000
E = 160000
D = 128
L = 4
NC = 10
NG = 1000
NUM_RUNS = 10
P = 2.0 / 11.0
NSEG = NUM_RUNS * N
ETOT = NUM_RUNS * E

NCORE = 2
NSUB = 16
RUNS_PC = NUM_RUNS // NCORE
SA = 624
G = 125
CPT = (E // NSUB) // G
IGRP = 40


def _stripe_copy(src_at, dst_at, s):
  @pl.when(s < NSUB - 1)
  def _():
    pltpu.sync_copy(src_at(s * SA, SA), dst_at(s * SA, SA))

  @pl.when(s == NSUB - 1)
  def _():
    pltpu.sync_copy(src_at((NSUB - 1) * SA, N - (NSUB - 1) * SA),
                    dst_at((NSUB - 1) * SA, N - (NSUB - 1) * SA))


def _sc_fast_body(xf_hbm, src2_hbm, dst2_hbm, out_hbm,
                  srcbuf, dstbuf, rba, rbb, acc, sema, semb):
  c = lax.axis_index("c")
  s = lax.axis_index("s")

  def gstart(k, buf, sem):
    pltpu.async_copy(xf_hbm.at[srcbuf.at[k]], buf, sem)

  def gwait(k, buf, sem):
    pltpu.make_async_copy(xf_hbm.at[srcbuf.at[k]], buf, sem).wait()

  def run_body(bi, _):
    r = bi * NCORE + c
    rbase = r * N

    _stripe_copy(lambda o, n: xf_hbm.at[pl.ds(rbase + o, n), :],
                 lambda o, n: acc.at[pl.ds(o, n), :], s)
    plsc.subcore_barrier()

    ib = r * (E // G) + s * CPT

    def igroup(gi, _):
      pltpu.sync_copy(src2_hbm.at[pl.ds(ib + gi * IGRP, IGRP), :], srcbuf)
      pltpu.sync_copy(dst2_hbm.at[pl.ds(ib + gi * IGRP, IGRP), :], dstbuf)

      gstart(0, rba, sema)

      def chunk2(t, _):
        k = 2 * t
        gstart(k + 1, rbb, semb)
        gwait(k, rba, sema)
        pltpu.sync_copy(rba, acc.at[dstbuf.at[k]], add=True)

        @pl.when(k + 2 < IGRP)
        def _():
          gstart(k + 2, rba, sema)
        gwait(k + 1, rbb, semb)
        pltpu.sync_copy(rbb, acc.at[dstbuf.at[k + 1]], add=True)
        return 0
      lax.fori_loop(0, IGRP // 2, chunk2, 0)
      return 0
    lax.fori_loop(0, CPT // IGRP, igroup, 0)

    plsc.subcore_barrier()
    _stripe_copy(lambda o, n: acc.at[pl.ds(o, n), :],
                 lambda o, n: out_hbm.at[pl.ds(rbase + o, n), :], s)
    return 0

  lax.fori_loop(0, RUNS_PC, run_body, 0)


_SC_MESH = plsc.VectorSubcoreMesh(core_axis_name="c", subcore_axis_name="s",
                                  num_cores=NCORE, num_subcores=NSUB)

_sc_fast = pl.kernel(
    _sc_fast_body,
    out_type=jax.ShapeDtypeStruct((NSEG, D), jnp.float32),
    mesh=_SC_MESH,
    scratch_types=[
        pltpu.VMEM((IGRP, G), jnp.int32),
        pltpu.VMEM((IGRP, G), jnp.int32),
        pltpu.VMEM((G, D), jnp.float32),
        pltpu.VMEM((G, D), jnp.float32),
        pltpu.VMEM_SHARED((N, D), jnp.float32),
        pltpu.SemaphoreType.DMA,
        pltpu.SemaphoreType.DMA,
    ],
)

RPB = 3200
NBPC = 16
FSTRIPE = RPB // NSUB
G2 = 128
FROWS = 12544
FRPT = FROWS // NSUB
FCB = 112
FCPT = FRPT // FCB


def _sc_gen_body(xf_hbm, src2_hbm, dst2_hbm, out_hbm,
                 srcbuf, dstbuf, rowbuf, acc, sem):
  c = lax.axis_index("c")
  s = lax.axis_index("s")

  def pass_body(bi, _):
    base = (bi * NCORE + c) * RPB
    slo = base + s * FSTRIPE

    @pl.when(slo + FSTRIPE <= NSEG)
    def _():
      pltpu.sync_copy(xf_hbm.at[pl.ds(slo, FSTRIPE), :],
                      acc.at[pl.ds(s * FSTRIPE, FSTRIPE), :])
    plsc.subcore_barrier()

    def block(t, _):
      ib = s * FRPT + t * FCB
      pltpu.sync_copy(src2_hbm.at[pl.ds(ib, FCB), :], srcbuf)
      pltpu.sync_copy(dst2_hbm.at[pl.ds(ib, FCB), :], dstbuf)

      def fix(k, _):
        def grp(j, _):
          sl = pl.ds(j * 16, 16)
          dv = dstbuf[k, sl]
          sv = srcbuf[k, sl]
          m = jnp.logical_and(dv >= base, dv < base + RPB)
          srcbuf[k, sl] = jnp.where(m, sv, jnp.int32(0))
          dstbuf[k, sl] = jnp.where(m, dv - base, jnp.int32(RPB))
          return 0
        lax.fori_loop(0, G2 // 16, grp, 0)
        return 0
      lax.fori_loop(0, FCB, fix, 0)

      def chunk(k, _):
        pltpu.async_copy(xf_hbm.at[srcbuf.at[k]], rowbuf, sem).wait()
        pltpu.sync_copy(rowbuf, acc.at[dstbuf.at[k]], add=True)
        return 0
      lax.fori_loop(0, FCB, chunk, 0)
      return 0
    lax.fori_loop(0, FCPT, block, 0)

    plsc.subcore_barrier()

    @pl.when(slo + FSTRIPE <= NSEG)
    def _():
      pltpu.sync_copy(acc.at[pl.ds(s * FSTRIPE, FSTRIPE), :],
                      out_hbm.at[pl.ds(slo, FSTRIPE), :])
    plsc.subcore_barrier()
    return 0

  lax.fori_loop(0, NBPC, pass_body, 0)


_sc_gen = pl.kernel(
    _sc_gen_body,
    out_type=jax.ShapeDtypeStruct((NSEG, D), jnp.float32),
    mesh=_SC_MESH,
    scratch_types=[
        pltpu.VMEM((FCB, G2), jnp.int32),
        pltpu.VMEM((FCB, G2), jnp.int32),
        pltpu.VMEM((G2, D), jnp.float32),
        pltpu.VMEM_SHARED((RPB + 8, D), jnp.float32),
        pltpu.SemaphoreType.DMA,
    ],
)


BR = 10000
NBLK = NSEG // BR
PBR = 5000
HP = jax.lax.Precision.DEFAULT


def _dot(a, b):
  return lax.dot_general(a, b, (((1,), (0,)), ((), ())),
                         precision=HP, preferred_element_type=jnp.float32)


def _k0_body(x_ref, keep_ref, fcw_ref, xr_ref, q_ref):
  pid = pl.program_id(0)
  xr = x_ref[...] * keep_ref[...]
  xr_ref[...] = xr
  contrib = _dot(xr, fcw_ref[...])

  @pl.when(pid == 0)
  def _():
    q_ref[...] = contrib

  @pl.when(pid > 0)
  def _():
    q_ref[...] = q_ref[...] + contrib


def _ka_body(h_ref, w_ref, b_ref, h1_ref, st_ref, acc):
  pid = pl.program_id(0)
  h1 = _dot(h_ref[...], w_ref[...]) + b_ref[...]
  h1_ref[...] = h1.astype(jnp.bfloat16)
  ssum = jnp.sum(h1, axis=0, keepdims=True)
  ssq = jnp.sum(h1 * h1, axis=0, keepdims=True)

  @pl.when(pid == 0)
  def _():
    acc[0:1, :] = ssum
    acc[1:2, :] = ssq

  @pl.when(pid > 0)
  def _():
    acc[0:1, :] = acc[0:1, :] + ssum
    acc[1:2, :] = acc[1:2, :] + ssq

  @pl.when(pid == NBLK - 1)
  def _():
    st_ref[...] = acc[...]


def _norm_relu(h, st, g, b):
  m = st[0:1, :] * (1.0 / NSEG)
  var = st[1:2, :] * (1.0 / NSEG) - m * m
  scale = lax.rsqrt(var + 1e-5) * g
  shift = b - m * scale
  return jnp.maximum(h * scale + shift, 0.0)


def _kb_body(h1_ref, st_ref, g_ref, bb_ref, w_ref, b2_ref, h2_ref, st2_ref,
             acc):
  pid = pl.program_id(0)
  hn = _norm_relu(h1_ref[...].astype(jnp.float32), st_ref[...], g_ref[...],
                  bb_ref[...])
  h2 = _dot(hn, w_ref[...]) + b2_ref[...]
  h2_ref[...] = h2.astype(jnp.bfloat16)
  ssum = jnp.sum(h2, axis=0, keepdims=True)
  ssq = jnp.sum(h2 * h2, axis=0, keepdims=True)

  @pl.when(pid == 0)
  def _():
    acc[0:1, :] = ssum
    acc[1:2, :] = ssq

  @pl.when(pid > 0)
  def _():
    acc[0:1, :] = acc[0:1, :] + ssum
    acc[1:2, :] = acc[1:2, :] + ssq

  @pl.when(pid == NBLK - 1)
  def _():
    st2_ref[...] = acc[...]


def _kc_body(h2_ref, st_ref, g_ref, bb_ref, fcw_ref, xf_ref, q_ref):
  pid = pl.program_id(0)
  xf = _norm_relu(h2_ref[...].astype(jnp.float32), st_ref[...], g_ref[...],
                  bb_ref[...])
  xf_ref[...] = xf
  contrib = _dot(xf, fcw_ref[...])

  @pl.when(pid == 0)
  def _():
    q_ref[...] = contrib

  @pl.when(pid > 0)
  def _():
    q_ref[...] = q_ref[...] + contrib


def _kf_body(q0, q1, q2, q3, q4, batch_ref, fcb_ref, out_ref, acc):
  pid = pl.program_id(0)
  qs = q0[...] + q1[...] + q2[...] + q3[...] + q4[...]
  bvals = batch_ref[0]
  giota = lax.broadcasted_iota(jnp.int32, (NG, PBR), 0)
  oh = jnp.where(bvals == giota, 1.0, 0.0)
  part = lax.dot_general(oh, qs, (((1,), (0,)), ((), ())),
                         precision=HP, preferred_element_type=jnp.float32)

  @pl.when(pid == 0)
  def _():
    acc[...] = part

  @pl.when(pid == 1)
  def _():
    z = (acc[...] + part) * (1.0 / NUM_RUNS)
    z = z + jnp.sum(fcb_ref[...], axis=0, keepdims=True)
    mx = jnp.max(z, axis=1, keepdims=True)
    zz = z - mx
    out_ref[...] = zz - jnp.log(jnp.sum(jnp.exp(zz), axis=1, keepdims=True))


_f32 = jnp.float32


def _spec(bs, im):
  return pl.BlockSpec(bs, im)


_k0 = pl.pallas_call(
    _k0_body,
    grid=(NBLK,),
    in_specs=[
        _spec((N, D), lambda i: (0, 0)),
        _spec((BR, 1), lambda i: (i, 0)),
        _spec((D, NC), lambda i: (0, 0)),
    ],
    out_specs=[
        _spec((BR, D), lambda i: (i, 0)),
        _spec((N, NC), lambda i: (0, 0)),
    ],
    out_shape=[
        jax.ShapeDtypeStruct((NSEG, D), _f32),
        jax.ShapeDtypeStruct((N, NC), _f32),
    ],
)

_ka = pl.pallas_call(
    _ka_body,
    grid=(NBLK,),
    in_specs=[
        _spec((BR, D), lambda i: (i, 0)),
        _spec((D, D), lambda i: (0, 0)),
        _spec((1, D), lambda i: (0, 0)),
    ],
    out_specs=[
        _spec((BR, D), lambda i: (i, 0)),
        _spec((2, D), lambda i: (0, 0)),
    ],
    out_shape=[
        jax.ShapeDtypeStruct((NSEG, D), jnp.bfloat16),
        jax.ShapeDtypeStruct((2, D), _f32),
    ],
    scratch_shapes=[pltpu.VMEM((2, D), _f32)],
)

_kb = pl.pallas_call(
    _kb_body,
    grid=(NBLK,),
    in_specs=[
        _spec((BR, D), lambda i: (i, 0)),
        _spec((2, D), lambda i: (0, 0)),
        _spec((1, D), lambda i: (0, 0)),
        _spec((1, D), lambda i: (0, 0)),
        _spec((D, D), lambda i: (0, 0)),
        _spec((1, D), lambda i: (0, 0)),
    ],
    out_specs=[
        _spec((BR, D), lambda i: (i, 0)),
        _spec((2, D), lambda i: (0, 0)),
    ],
    out_shape=[
        jax.ShapeDtypeStruct((NSEG, D), jnp.bfloat16),
        jax.ShapeDtypeStruct((2, D), _f32),
    ],
    scratch_shapes=[pltpu.VMEM((2, D), _f32)],
)

_kc = pl.pallas_call(
    _kc_body,
    grid=(NBLK,),
    in_specs=[
        _spec((BR, D), lambda i: (i, 0)),
        _spec((2, D), lambda i: (0, 0)),
        _spec((1, D), lambda i: (0, 0)),
        _spec((1, D), lambda i: (0, 0)),
        _spec((D, NC), lambda i: (0, 0)),
    ],
    out_specs=[
        _spec((BR, D), lambda i: (i, 0)),
        _spec((N, NC), lambda i: (0, 0)),
    ],
    out_shape=[
        jax.ShapeDtypeStruct((NSEG, D), _f32),
        jax.ShapeDtypeStruct((N, NC), _f32),
    ],
)

_kf = pl.pallas_call(
    _kf_body,
    grid=(2,),
    in_specs=[
        _spec((PBR, NC), lambda i: (i, 0)),
        _spec((PBR, NC), lambda i: (i, 0)),
        _spec((PBR, NC), lambda i: (i, 0)),
        _spec((PBR, NC), lambda i: (i, 0)),
        _spec((PBR, NC), lambda i: (i, 0)),
        _spec((1, 1, PBR), lambda i: (i, 0, 0)),
        _spec((L + 1, NC), lambda i: (0, 0)),
    ],
    out_specs=_spec((NG, NC), lambda i: (0, 0)),
    out_shape=jax.ShapeDtypeStruct((NG, NC), _f32),
    scratch_shapes=[pltpu.VMEM((NG, NC), _f32)],
)


def kernel(x, edge_index, batch, convW1, convb1, conv_bn_g, conv_bn_b,
           convW2, convb2, bn_g, bn_b, fc_W, fc_b):
  drop = jax.random.bernoulli(jax.random.key(42), P, (NUM_RUNS, N))
  keep = jnp.where(drop, 0.0, 1.0).astype(jnp.float32).reshape(NSEG, 1)
  offset = edge_index.max() + 1
  run_off = jnp.arange(NUM_RUNS, dtype=edge_index.dtype) * offset
  srcf = (edge_index[0][None, :] + run_off[:, None]).reshape(-1)
  dstf = (edge_index[1][None, :] + run_off[:, None]).reshape(-1)
  src2 = srcf.reshape(ETOT // G, G)
  dstl2 = jnp.broadcast_to(edge_index[1][None, :],
                           (NUM_RUNS, E)).reshape(ETOT // G, G)
  npad = FROWS * G2 - ETOT
  src2g = jnp.concatenate(
      [srcf, jnp.zeros((npad,), jnp.int32)]).reshape(FROWS, G2)
  dst2g = jnp.concatenate(
      [dstf, jnp.full((npad,), -1, jnp.int32)]).reshape(FROWS, G2)

  def segsum(xf):
    return lax.cond(offset == N,
                    lambda a: _sc_fast(a, src2, dstl2),
                    lambda a: _sc_gen(a, src2g, dst2g),
                    xf)

  xr, q0 = _k0(x, keep, fc_W[0])
  qs = [q0]
  xf = xr
  for i in range(L):
    hsum = segsum(xf)
    h1, st1 = _ka(hsum, convW1[i], convb1[i][None, :])
    h2, st2 = _kb(h1, st1, conv_bn_g[i][None, :], conv_bn_b[i][None, :],
                  convW2[i], convb2[i][None, :])
    xf, q = _kc(h2, st2, bn_g[i][None, :], bn_b[i][None, :], fc_W[i + 1])
    qs.append(q)
  return _kf(*qs, batch.reshape(2, 1, PBR), fc_b)
